# baseline (speedup 1.0000x reference)
import functools

import jax
import jax.numpy as jnp
from jax import lax
from jax.experimental import pallas as pl
from jax.experimental.pallas import tpu as pltpu
from jax.experimental.pallas import tpu_sc as plsc

_B = 16384
_D = 64
_NC, _NS = 2, 16
_NW = _NC * _NS
_RPW = _B // _NW
_CHUNK = 128
_NCH = _RPW // _CHUNK


def _gather_body(user_r, item_r, mfu, mfi, mlu, mli,
                 o_mfu, o_mfi, o_mlu, o_mli,
                 idx_u, idx_i, buf_a, buf_b, sem_a, sem_b):
    wid = lax.axis_index("s") * _NC + lax.axis_index("c")
    base = wid * _RPW
    pltpu.sync_copy(user_r.at[wid], idx_u)
    pltpu.sync_copy(item_r.at[wid], idx_i)
    for tab_u, tab_i, out_u, out_i in ((mfu, mfi, o_mfu, o_mfi),
                                       (mlu, mli, o_mlu, o_mli)):
        ga = [pltpu.async_copy(tab_u.at[idx_u.at[j]], buf_a.at[j], sem_a)
              for j in range(_NCH)]
        gb = [pltpu.async_copy(tab_i.at[idx_i.at[j]], buf_b.at[j], sem_b)
              for j in range(_NCH)]
        for j in range(_NCH):
            ga[j].wait()
            pltpu.sync_copy(buf_a.at[j],
                            out_u.at[pl.ds(base + j * _CHUNK, _CHUNK)])
        for j in range(_NCH):
            gb[j].wait()
            pltpu.sync_copy(buf_b.at[j],
                            out_i.at[pl.ds(base + j * _CHUNK, _CHUNK)])


_gather = functools.partial(
    pl.kernel,
    out_type=[jax.ShapeDtypeStruct((_B, _D), jnp.float32)] * 4,
    mesh=plsc.VectorSubcoreMesh(core_axis_name="c", subcore_axis_name="s"),
    scratch_types=[
        pltpu.VMEM((_NCH, _CHUNK), jnp.int32),
        pltpu.VMEM((_NCH, _CHUNK), jnp.int32),
        pltpu.VMEM((_NCH, _CHUNK, _D), jnp.float32),
        pltpu.VMEM((_NCH, _CHUNK, _D), jnp.float32),
        pltpu.SemaphoreType.DMA,
        pltpu.SemaphoreType.DMA,
    ],
    compiler_params=pltpu.CompilerParams(use_tc_tiling_on_sc=False),
)(_gather_body)


_BLK = 2048


def _mlp_body(xmfu_ref, xmfi_ref, xu_ref, xi_ref,
              w1_ref, b1_ref, w2_ref, b2_ref, w3_ref, b3_ref,
              wf_ref, bf_ref, out_ref):
    dn = (((1,), (1,)), ((), ()))
    f32 = jnp.float32
    w1 = w1_ref[...]
    h = lax.dot_general(xu_ref[...], w1[:, :_D], dn, preferred_element_type=f32)
    h = h + lax.dot_general(xi_ref[...], w1[:, _D:], dn, preferred_element_type=f32)
    h = jnp.maximum(h + b1_ref[...], 0.0)
    h = lax.dot_general(h, w2_ref[...], dn, preferred_element_type=f32)
    h = jnp.maximum(h + b2_ref[...], 0.0)
    h = lax.dot_general(h, w3_ref[...], dn, preferred_element_type=f32)
    h = jnp.maximum(h + b3_ref[...], 0.0)
    xmf = xmfu_ref[...] * xmfi_ref[...]
    wf = wf_ref[...]
    logit = lax.dot_general(xmf, wf[:, :_D], dn, preferred_element_type=f32)
    logit = logit + lax.dot_general(h, wf[:, _D:], dn, preferred_element_type=f32)
    out_ref[...] = jax.nn.sigmoid(logit + bf_ref[...])


def kernel(user, item, mf_user_embed, mf_item_embed, mlp_user_embed,
           mlp_item_embed, W1, b1, W2, b2, W3, b3, Wf, bf):
    user_r = user.reshape(_NW, _NCH, _CHUNK)
    item_r = item.reshape(_NW, _NCH, _CHUNK)
    xmfu, xmfi, xu, xi = _gather(user_r, item_r, mf_user_embed, mf_item_embed,
                                 mlp_user_embed, mlp_item_embed)
    full = lambda shape: pl.BlockSpec(shape, lambda i: (0,) * len(shape))
    out = pl.pallas_call(
        _mlp_body,
        grid=(_B // _BLK,),
        in_specs=[
            pl.BlockSpec((_BLK, _D), lambda i: (i, 0)),
            pl.BlockSpec((_BLK, _D), lambda i: (i, 0)),
            pl.BlockSpec((_BLK, _D), lambda i: (i, 0)),
            pl.BlockSpec((_BLK, _D), lambda i: (i, 0)),
            full((64, 128)), full((1, 64)),
            full((32, 64)), full((1, 32)),
            full((16, 32)), full((1, 16)),
            full((1, 80)), full((1, 1)),
        ],
        out_specs=pl.BlockSpec((_BLK, 1), lambda i: (i, 0)),
        out_shape=jax.ShapeDtypeStruct((_B, 1), jnp.float32),
    )(xmfu, xmfi, xu, xi,
      W1, b1.reshape(1, 64), W2, b2.reshape(1, 32), W3, b3.reshape(1, 16),
      Wf, bf.reshape(1, 1))
    return out


# speedup vs baseline: 1.0000x; 1.0000x over previous
import functools

import jax
import jax.numpy as jnp
from jax import lax
from jax.experimental import pallas as pl
from jax.experimental.pallas import tpu as pltpu
from jax.experimental.pallas import tpu_sc as plsc

_B = 16384
_D = 64
_NC, _NS = 2, 16
_NW = _NC * _NS
_RPW = _B // _NW
_CHUNK = 128
_NCH = _RPW // _CHUNK
_NTASK = 4 * _NCH
_DEPTH = 6


def _gather_body(u2_hbm, i2_hbm, mfu, mfi, mlu, mli,
                 o_mfu, o_mfi, o_mlu, o_mli,
                 idx_u, idx_i, buf, gsems, wsems):
    wid = lax.axis_index("s") * _NC + lax.axis_index("c")
    base = wid * _RPW
    pltpu.sync_copy(u2_hbm.at[pl.ds(base, _RPW)], idx_u)
    pltpu.sync_copy(i2_hbm.at[pl.ds(base, _RPW)], idx_i)
    tasks = []
    for tab, idx, out in ((mfu, idx_u, o_mfu), (mfi, idx_i, o_mfi),
                          (mlu, idx_u, o_mlu), (mli, idx_i, o_mli)):
        for j in range(_NCH):
            tasks.append((tab, idx, out, j))
    gh = [None] * _DEPTH
    wh = [None] * _DEPTH
    for q in range(_NTASK + _DEPTH):
        if q >= _DEPTH:
            qq = q - _DEPTH
            k = qq % _DEPTH
            _, _, out, j = tasks[qq]
            gh[k].wait()
            wh[k] = pltpu.async_copy(
                buf.at[k], out.at[pl.ds(base + j * _CHUNK, _CHUNK)],
                wsems.at[k])
        if q < _NTASK:
            k = q % _DEPTH
            tab, idx, out, j = tasks[q]
            if wh[k] is not None:
                wh[k].wait()
            gh[k] = pltpu.async_copy(
                tab.at[idx.at[pl.ds(j * _CHUNK, _CHUNK)]], buf.at[k],
                gsems.at[k])
    for q in range(_NTASK - _DEPTH, _NTASK):
        wh[q % _DEPTH].wait()


_gather = functools.partial(
    pl.kernel,
    out_type=[jax.ShapeDtypeStruct((_B, 2 * _D), jnp.float32)] * 4,
    mesh=plsc.VectorSubcoreMesh(core_axis_name="c", subcore_axis_name="s"),
    scratch_types=[
        pltpu.VMEM((_RPW,), jnp.int32),
        pltpu.VMEM((_RPW,), jnp.int32),
        pltpu.VMEM((_DEPTH, _CHUNK, 2 * _D), jnp.float32),
        pltpu.SemaphoreType.DMA((_DEPTH,)),
        pltpu.SemaphoreType.DMA((_DEPTH,)),
    ],
)(_gather_body)


_BLK = 2048


def _mlp_body(pu_ref, pi_ref, xmfu_ref, xmfi_ref, xu_ref, xi_ref,
              w1_ref, b1_ref, w2_ref, b2_ref, w3_ref, b3_ref,
              wf_ref, bf_ref, out_ref):
    dn = (((1,), (1,)), ((), ()))
    f32 = jnp.float32
    pu = pu_ref[...] == 1
    pi = pi_ref[...] == 1

    def half(ref, p):
        x = ref[...]
        return jnp.where(p, x[:, _D:], x[:, :_D])

    xmfu = half(xmfu_ref, pu)
    xmfi = half(xmfi_ref, pi)
    xu = half(xu_ref, pu)
    xi = half(xi_ref, pi)
    w1 = w1_ref[...]
    h = lax.dot_general(xu, w1[:, :_D], dn, preferred_element_type=f32)
    h = h + lax.dot_general(xi, w1[:, _D:], dn, preferred_element_type=f32)
    h = jnp.maximum(h + b1_ref[...], 0.0)
    h = lax.dot_general(h, w2_ref[...], dn, preferred_element_type=f32)
    h = jnp.maximum(h + b2_ref[...], 0.0)
    h = lax.dot_general(h, w3_ref[...], dn, preferred_element_type=f32)
    h = jnp.maximum(h + b3_ref[...], 0.0)
    xmf = xmfu * xmfi
    wf = wf_ref[...]
    logit = lax.dot_general(xmf, wf[:, :_D], dn, preferred_element_type=f32)
    logit = logit + lax.dot_general(h, wf[:, _D:], dn, preferred_element_type=f32)
    out_ref[...] = jax.nn.sigmoid(logit + bf_ref[...])


def kernel(user, item, mf_user_embed, mf_item_embed, mlp_user_embed,
           mlp_item_embed, W1, b1, W2, b2, W3, b3, Wf, bf):
    user2 = jnp.right_shift(user, 1)
    item2 = jnp.right_shift(item, 1)
    pu = jnp.bitwise_and(user, 1).reshape(_B, 1)
    pi = jnp.bitwise_and(item, 1).reshape(_B, 1)
    pair = lambda t: t.reshape(t.shape[0] // 2, 2 * _D)
    xmfu2, xmfi2, xu2, xi2 = _gather(
        user2, item2, pair(mf_user_embed), pair(mf_item_embed),
        pair(mlp_user_embed), pair(mlp_item_embed))
    full = lambda shape: pl.BlockSpec(shape, lambda i: (0,) * len(shape))
    row = lambda w: pl.BlockSpec((_BLK, w), lambda i: (i, 0))
    out = pl.pallas_call(
        _mlp_body,
        grid=(_B // _BLK,),
        in_specs=[
            row(1), row(1),
            row(2 * _D), row(2 * _D), row(2 * _D), row(2 * _D),
            full((64, 128)), full((1, 64)),
            full((32, 64)), full((1, 32)),
            full((16, 32)), full((1, 16)),
            full((1, 80)), full((1, 1)),
        ],
        out_specs=pl.BlockSpec((_BLK, 1), lambda i: (i, 0)),
        out_shape=jax.ShapeDtypeStruct((_B, 1), jnp.float32),
    )(pu, pi, xmfu2, xmfi2, xu2, xi2,
      W1, b1.reshape(1, 64), W2, b2.reshape(1, 32), W3, b3.reshape(1, 16),
      Wf, bf.reshape(1, 1))
    return out


# speedup vs baseline: 1.0047x; 1.0047x over previous
import functools

import jax
import jax.numpy as jnp
from jax import lax
from jax.experimental import pallas as pl
from jax.experimental.pallas import tpu as pltpu
from jax.experimental.pallas import tpu_sc as plsc

_B = 16384
_D = 64
_NC, _NS = 2, 16
_NW = _NC * _NS
_RPW = _B // _NW
_G = 16
_NG = _RPW // _G


def _gather_body(idx_hbm, tab, out, idx_v, buf, gsem, wsem):
    wid = lax.axis_index("s") * _NC + lax.axis_index("c")
    base = wid * _RPW
    pltpu.sync_copy(idx_hbm.at[pl.ds(base, _RPW)], idx_v)

    def fire(j, _):
        iv = idx_v[pl.ds(j * _G, _G)]
        pltpu.async_copy(tab.at[iv], buf.at[pl.ds(j * _G, _G)], gsem)
        return 0

    lax.fori_loop(0, _NG, fire, 0)

    def drain(j, _):
        pltpu.make_async_copy(tab.at[pl.ds(0, _G)], buf.at[pl.ds(0, _G)],
                              gsem).wait()
        return 0

    lax.fori_loop(0, _NG, drain, 0)
    pltpu.async_copy(buf, out.at[pl.ds(base, _RPW)], wsem).wait()


_gather1 = functools.partial(
    pl.kernel,
    out_type=jax.ShapeDtypeStruct((_B, _D), jnp.float32),
    mesh=plsc.VectorSubcoreMesh(core_axis_name="c", subcore_axis_name="s"),
    scratch_types=[
        pltpu.VMEM((_RPW,), jnp.int32),
        pltpu.VMEM((_RPW, _D), jnp.float32),
        pltpu.SemaphoreType.DMA,
        pltpu.SemaphoreType.DMA,
    ],
    compiler_params=pltpu.CompilerParams(use_tc_tiling_on_sc=False),
)(_gather_body)


_BLK = 2048


def _mlp_body(xmfu_ref, xmfi_ref, xu_ref, xi_ref,
              w1_ref, b1_ref, w2_ref, b2_ref, w3_ref, b3_ref,
              wf_ref, bf_ref, out_ref):
    dn = (((1,), (1,)), ((), ()))
    f32 = jnp.float32
    w1 = w1_ref[...]
    h = lax.dot_general(xu_ref[...], w1[:, :_D], dn, preferred_element_type=f32)
    h = h + lax.dot_general(xi_ref[...], w1[:, _D:], dn, preferred_element_type=f32)
    h = jnp.maximum(h + b1_ref[...], 0.0)
    h = lax.dot_general(h, w2_ref[...], dn, preferred_element_type=f32)
    h = jnp.maximum(h + b2_ref[...], 0.0)
    h = lax.dot_general(h, w3_ref[...], dn, preferred_element_type=f32)
    h = jnp.maximum(h + b3_ref[...], 0.0)
    xmf = xmfu_ref[...] * xmfi_ref[...]
    wf = wf_ref[...]
    logit = lax.dot_general(xmf, wf[:, :_D], dn, preferred_element_type=f32)
    logit = logit + lax.dot_general(h, wf[:, _D:], dn, preferred_element_type=f32)
    out_ref[...] = jax.nn.sigmoid(logit + bf_ref[...])


def kernel(user, item, mf_user_embed, mf_item_embed, mlp_user_embed,
           mlp_item_embed, W1, b1, W2, b2, W3, b3, Wf, bf):
    xmfu = _gather1(user, mf_user_embed)
    xmfi = _gather1(item, mf_item_embed)
    xu = _gather1(user, mlp_user_embed)
    xi = _gather1(item, mlp_item_embed)
    full = lambda shape: pl.BlockSpec(shape, lambda i: (0,) * len(shape))
    row = lambda w: pl.BlockSpec((_BLK, w), lambda i: (i, 0))
    out = pl.pallas_call(
        _mlp_body,
        grid=(_B // _BLK,),
        in_specs=[
            row(_D), row(_D), row(_D), row(_D),
            full((64, 128)), full((1, 64)),
            full((32, 64)), full((1, 32)),
            full((16, 32)), full((1, 16)),
            full((1, 80)), full((1, 1)),
        ],
        out_specs=pl.BlockSpec((_BLK, 1), lambda i: (i, 0)),
        out_shape=jax.ShapeDtypeStruct((_B, 1), jnp.float32),
    )(xmfu, xmfi, xu, xi,
      W1, b1.reshape(1, 64), W2, b2.reshape(1, 32), W3, b3.reshape(1, 16),
      Wf, bf.reshape(1, 1))
    return out
